# Initial kernel scaffold; baseline (speedup 1.0000x reference)
import jax
import jax.numpy as jnp
from jax.experimental import pallas as pl


def kernel(x, router_logits, w13_weight, w2_weight, w13_weight_scale_inv, w2_weight_scale_inv, top_k, renormalize):
    raise NotImplementedError("write your pallas kernel here")


# speedup vs baseline: 4.1309x; 4.1309x over previous
import functools

import jax
import jax.numpy as jnp
from jax import lax
from jax.experimental import pallas as pl
from jax.experimental.pallas import tpu as pltpu

_T, _H, _I, _E = 2048, 1024, 2048, 8
_BN = 128
_K = 2
_BM = 256
_M = _T * _K
_NB = _M // _BM + _E
_NBM = _NB * _BM


def _dequant_body(w_ref, s_ref, o_ref):
    rows = w_ref.shape[1]
    for r in range(rows // _BN):
        o_ref[0, r * _BN:(r + 1) * _BN, :] = (
            w_ref[0, r * _BN:(r + 1) * _BN, :] * s_ref[0, r:r + 1, :]
        ).astype(jnp.bfloat16)


def _dequant13(w13, s13k):
    return pl.pallas_call(
        _dequant_body,
        grid=(_E, 2),
        in_specs=[
            pl.BlockSpec((1, _I, _H), lambda e, c: (e, c, 0)),
            pl.BlockSpec((1, _I // _BN, _H), lambda e, c: (e, c, 0)),
        ],
        out_specs=pl.BlockSpec((1, _I, _H), lambda e, c: (e, c, 0)),
        out_shape=jax.ShapeDtypeStruct((_E, 2 * _I, _H), jnp.bfloat16),
    )(w13, s13k)


def _dequant2(w2, s2k):
    return pl.pallas_call(
        _dequant_body,
        grid=(_E,),
        in_specs=[
            pl.BlockSpec((1, _H, _I), lambda e: (e, 0, 0)),
            pl.BlockSpec((1, _H // _BN, _I), lambda e: (e, 0, 0)),
        ],
        out_specs=pl.BlockSpec((1, _H, _I), lambda e: (e, 0, 0)),
        out_shape=jax.ShapeDtypeStruct((_E, _H, _I), jnp.bfloat16),
    )(w2, s2k)


def _ffn_body(be_ref, xs_ref, w13_ref, w2_ref, g_ref, o_ref):
    x = xs_ref[...]
    h = lax.dot_general(x, w13_ref[0], (((1,), (1,)), ((), ())),
                        preferred_element_type=jnp.float32)
    gate = h[:, :_I]
    up = h[:, _I:]
    act = (gate * jax.nn.sigmoid(gate) * up).astype(jnp.bfloat16)
    o = lax.dot_general(act, w2_ref[0], (((1,), (1,)), ((), ())),
                        preferred_element_type=jnp.float32)
    o_ref[...] = o * g_ref[...]


def _ffn(block_expert, xs, w13f, w2f, gamma):
    grid_spec = pltpu.PrefetchScalarGridSpec(
        num_scalar_prefetch=1,
        grid=(_NB,),
        in_specs=[
            pl.BlockSpec((_BM, _H), lambda i, be: (i, 0)),
            pl.BlockSpec((1, 2 * _I, _H), lambda i, be: (be[i], 0, 0)),
            pl.BlockSpec((1, _H, _I), lambda i, be: (be[i], 0, 0)),
            pl.BlockSpec((_BM, 1), lambda i, be: (i, 0)),
        ],
        out_specs=pl.BlockSpec((_BM, _H), lambda i, be: (i, 0)),
    )
    return pl.pallas_call(
        _ffn_body,
        grid_spec=grid_spec,
        out_shape=jax.ShapeDtypeStruct((_NBM, _H), jnp.float32),
    )(block_expert, xs, w13f, w2f, gamma)


def kernel(x, router_logits, w13_weight, w2_weight, w13_weight_scale_inv,
           w2_weight_scale_inv, top_k, renormalize):
    probs = jax.nn.softmax(router_logits.astype(jnp.float32), axis=-1)
    tw, ti = lax.top_k(probs, _K)
    tw = tw * (jnp.asarray(top_k, jnp.float32) / _K)
    tw = jnp.where(jnp.asarray(renormalize) != 0,
                   tw / jnp.sum(tw, axis=-1, keepdims=True), tw)

    flat_ids = ti.reshape(-1).astype(jnp.int32)
    oh = flat_ids[:, None] == jnp.arange(_E, dtype=jnp.int32)[None, :]
    ohi = oh.astype(jnp.int32)
    counts = ohi.sum(axis=0)
    rank = jnp.where(oh, jnp.cumsum(ohi, axis=0) - 1, 0).sum(axis=1)
    nblk = (counts + _BM - 1) // _BM
    bstart = jnp.concatenate(
        [jnp.zeros((1,), jnp.int32), jnp.cumsum(nblk)[:-1].astype(jnp.int32)])
    dest = bstart[flat_ids] * _BM + rank
    token = jnp.arange(_M, dtype=jnp.int32) // _K
    sorted_token = jnp.zeros((_NBM,), jnp.int32).at[dest].set(token)
    gamma = jnp.zeros((_NBM, 1), jnp.float32).at[dest, 0].set(tw.reshape(-1))
    block_expert = jnp.repeat(jnp.arange(_E, dtype=jnp.int32), nblk,
                              total_repeat_length=_NB).astype(jnp.int32)

    s13k = jnp.repeat(w13_weight_scale_inv, _BN, axis=2)
    s2k = jnp.repeat(w2_weight_scale_inv, _BN, axis=2)
    w13f = _dequant13(w13_weight, s13k)
    w2f = _dequant2(w2_weight, s2k)

    xs = x.astype(jnp.bfloat16)[sorted_token]
    o_sorted = _ffn(block_expert, xs, w13f, w2f, gamma)
    d = dest.reshape(_T, _K)
    return o_sorted[d[:, 0]] + o_sorted[d[:, 1]]
